# baseline (speedup 1.0000x reference)
import math

import jax
import jax.numpy as jnp
from jax.experimental import pallas as pl
from jax.experimental.pallas import tpu as pltpu

B = 16
L = 576
SIZE = 24
D = 768
NUM_PARTS = 4
PARTS_DIM = D // NUM_PARTS
PART_CHANNELS = 32
GCN_DIM = 392
NUM_EDGES = NUM_PARTS * (NUM_PARTS - 1) * B
N_NODES = B * NUM_PARTS
BB = 2
PHA = B // BB
HID = PARTS_DIM // 16
CPS = 4
NKB = PART_CHANNELS // CPS
NTOT = PHA + NKB
PC4 = NUM_PARTS * PART_CHANNELS

_INV_SQRT2 = 1.0 / math.sqrt(2.0)


def _gelu(t):
    return 0.5 * t * (1.0 + jax.lax.erf(t * _INV_SQRT2))


def _ln(t, g, b, eps=1e-6):
    m = jnp.mean(t, axis=-1, keepdims=True)
    v = jnp.mean((t - m) ** 2, axis=-1, keepdims=True)
    return (t - m) / jnp.sqrt(v + eps) * g + b


def _dot(a, b):
    return jnp.dot(a, b, preferred_element_type=jnp.float32)


def _build_A(ei_ref, eit_ref, ews_ref, ewst_ref, blk):
    ew_row = jax.nn.sigmoid(ews_ref[blk:blk + 1, :])
    ew_col = jax.nn.sigmoid(ewst_ref[:, blk:blk + 1])
    dst_row = ei_ref[1:2, :]
    src_col = eit_ref[:, 0:1]
    dst_col = eit_ref[:, 1:2]
    row_ids = jax.lax.broadcasted_iota(jnp.int32, (N_NODES, NUM_EDGES), 0)
    col_ids = jax.lax.broadcasted_iota(jnp.int32, (NUM_EDGES, N_NODES), 1)
    Mdst = jnp.where(row_ids == dst_row, 1.0, 0.0)
    MdstT = jnp.where(col_ids == dst_col, 1.0, 0.0)
    Msrc = jnp.where(col_ids == src_col, 1.0, 0.0)
    deg_col = _dot(Mdst, ew_col) + 1.0
    deg_row = _dot(ew_row, MdstT) + 1.0
    dis_col = jnp.where(deg_col > 0,
                        jax.lax.rsqrt(jnp.maximum(deg_col, 1e-12)), 0.0)
    dis_row = jnp.where(deg_row > 0,
                        jax.lax.rsqrt(jnp.maximum(deg_row, 1e-12)), 0.0)
    W_raw = _dot(Mdst * ew_row, Msrc)
    ir = jax.lax.broadcasted_iota(jnp.int32, (N_NODES, N_NODES), 0)
    ic = jax.lax.broadcasted_iota(jnp.int32, (N_NODES, N_NODES), 1)
    eye = jnp.where(ir == ic, 1.0, 0.0)
    return dis_col * (W_raw + eye) * dis_row


def _fused_body(x_ref, dm_ref, W1bd_ref, W2bd_ref, ws_ref, bs_ref, Wpe_ref,
                e32_ref, bp_ref, fw_ref, fb_ref, ei_ref, eit_ref, ews_ref,
                ewst_ref,
                W1a_ref, b1a_ref, g1a_ref, be1a_ref,
                W2a_ref, b2a_ref, g2a_ref, be2a_ref,
                W1b_ref, b1b_ref, g1b_ref, be1b_ref,
                W2b_ref, b2b_ref, g2b_ref, be2b_ref,
                Wd_ref, gd_ref, bd_ref,
                m_ref, out_ref, x3_scr, sh_scr, acc_ref):
    i = pl.program_id(0)

    @pl.when(i == 0)
    def _():
        acc_ref[...] = jnp.zeros_like(acc_ref)

    @pl.when(i < PHA)
    def _():
        for bb in range(BB):
            xf = x_ref[bb]
            dmv = dm_ref[bb]
            avg_all = jnp.mean(xf, axis=0, keepdims=True)
            mx_all = jnp.max(xf, axis=0, keepdims=True)
            ha = jnp.maximum(_dot(avg_all, W1bd_ref[...]), 0.0)
            hm = jnp.maximum(_dot(mx_all, W1bd_ref[...]), 0.0)
            gate = jax.nn.sigmoid(_dot(ha, W2bd_ref[...])
                                  + _dot(hm, W2bd_ref[...]))
            xg = xf * gate
            z = _dot(xg, Wpe_ref[...])
            savg4 = z[:, PC4:]
            smax4 = jnp.concatenate(
                [jnp.max(xg[:, p * PARTS_DIM:(p + 1) * PARTS_DIM],
                         axis=1, keepdims=True) for p in range(NUM_PARTS)],
                axis=1)
            sm4 = jax.nn.sigmoid(savg4 * ws_ref[0:1] + smax4 * ws_ref[1:2]
                                 + dmv * ws_ref[2:3] + bs_ref[...])
            sm128 = _dot(sm4, e32_ref[...])
            xo = z[:, :PC4] * sm128 + bp_ref[...]
            xoT = xo.T
            for p in range(NUM_PARTS):
                x3_scr[i, :, bb * NUM_PARTS + p, :] = (
                    xoT[p * PART_CHANNELS:(p + 1) * PART_CHANNELS])
            m_ref[bb * NUM_PARTS:(bb + 1) * NUM_PARTS] = sm4.T
            sh_scr[pl.ds(i * BB + bb, 1)] = avg_all

    @pl.when(i >= PHA)
    def _():
        c0 = (i - PHA) * CPS
        xgT = jnp.concatenate(
            [jnp.concatenate([x3_scr[s, c0 + j] for s in range(PHA)],
                             axis=0).T for j in range(CPS)], axis=0)
        acc_ref[...] += _dot(fw_ref[...], xgT)

    @pl.when(i == NTOT - 1)
    def _():
        y = acc_ref[...].T + fb_ref[...]
        A1 = _build_A(ei_ref, eit_ref, ews_ref, ewst_ref, 0)
        A2 = _build_A(ei_ref, eit_ref, ews_ref, ewst_ref, 1)

        h = _dot(A1, _dot(y, W1a_ref[...])) + b1a_ref[...]
        h = _gelu(_ln(h, g1a_ref[...], be1a_ref[...]))
        h = _dot(A1, _dot(h, W2a_ref[...])) + b2a_ref[...]
        y1 = _gelu(_ln(h, g2a_ref[...], be2a_ref[...]) + y)

        h = _dot(A2, _dot(y1, W1b_ref[...])) + b1b_ref[...]
        h = _gelu(_ln(h, g1b_ref[...], be1b_ref[...]))
        h = _dot(A2, _dot(h, W2b_ref[...])) + b2b_ref[...]
        h = _ln(h, g2b_ref[...], be2b_ref[...])
        sc = _ln(_dot(y1, Wd_ref[...]), gd_ref[...], bd_ref[...])
        y2 = _gelu(h + sc)

        pr = jax.lax.broadcasted_iota(jnp.int32, (B, N_NODES), 0)
        pc = jax.lax.broadcasted_iota(jnp.int32, (B, N_NODES), 1)
        pool = jnp.where(pc // NUM_PARTS == pr, 0.25, 0.0)
        out_ref[...] = _dot(pool, y2) + sh_scr[...]


@jax.jit
def kernel(decision_masks, x, params, edge_index):
    cb = params['cbam']
    W1bd = jax.scipy.linalg.block_diag(*[c['W1'] for c in cb])
    W2bd = jax.scipy.linalg.block_diag(*[c['W2'] for c in cb])
    Wpbd = jax.scipy.linalg.block_diag(*[c['Wp'] for c in cb])
    part_of_d = jnp.arange(D, dtype=jnp.int32) // PARTS_DIM
    bandmask = jnp.where(part_of_d[:, None]
                         == jnp.arange(NUM_PARTS, dtype=jnp.int32)[None, :],
                         1.0 / PARTS_DIM, 0.0)
    Wpe = jnp.concatenate([Wpbd, bandmask], axis=1)
    part_of_c = jnp.arange(PC4, dtype=jnp.int32) // PART_CHANNELS
    e32 = jnp.where(jnp.arange(NUM_PARTS, dtype=jnp.int32)[:, None]
                    == part_of_c[None, :], 1.0, 0.0)
    bp_row = jnp.concatenate([c['bp'] for c in cb]).reshape(1, PC4)
    ws_cols = jnp.stack([c['Ws'] for c in cb], axis=1)
    bs_row = jnp.stack([c['bs'] for c in cb]).reshape(1, NUM_PARTS)

    blocks = params['blocks']
    ews = jnp.stack([bp['edge_weight'] for bp in blocks])
    ewst = ews.T
    ei = edge_index.astype(jnp.int32)
    eit = ei.T
    b0, b1 = blocks

    full = lambda s: pl.BlockSpec(s, lambda i: tuple(0 for _ in s))
    r2 = lambda a: a.reshape(1, -1)

    masks_t, out = pl.pallas_call(
        _fused_body,
        grid=(NTOT,),
        in_specs=[
            pl.BlockSpec((BB, L, D), lambda i: (jnp.minimum(i, PHA - 1), 0, 0)),
            pl.BlockSpec((BB, L, 1), lambda i: (jnp.minimum(i, PHA - 1), 0, 0)),
            full((D, NUM_PARTS * HID)),
            full((NUM_PARTS * HID, D)),
            full((3, NUM_PARTS)),
            full((1, NUM_PARTS)),
            full((D, PC4 + NUM_PARTS)),
            full((NUM_PARTS, PC4)),
            full((1, PC4)),
            pl.BlockSpec((GCN_DIM, CPS * L),
                         lambda i: (0, jnp.maximum(i - PHA, 0))),
            full((1, GCN_DIM)),
            full((2, NUM_EDGES)),
            full((NUM_EDGES, 2)),
            full((2, NUM_EDGES)),
            full((NUM_EDGES, 2)),
            full(b0['W1'].shape), full((1, b0['b1'].shape[0])),
            full((1, b0['g1'].shape[0])), full((1, b0['be1'].shape[0])),
            full(b0['W2'].shape), full((1, b0['b2'].shape[0])),
            full((1, b0['g2'].shape[0])), full((1, b0['be2'].shape[0])),
            full(b1['W1'].shape), full((1, b1['b1'].shape[0])),
            full((1, b1['g1'].shape[0])), full((1, b1['be1'].shape[0])),
            full(b1['W2'].shape), full((1, b1['b2'].shape[0])),
            full((1, b1['g2'].shape[0])), full((1, b1['be2'].shape[0])),
            full(b1['Wd'].shape), full((1, b1['gd'].shape[0])),
            full((1, b1['bd'].shape[0])),
        ],
        out_specs=[
            pl.BlockSpec((BB * NUM_PARTS, L),
                         lambda i: (jnp.minimum(i, PHA - 1), 0)),
            pl.BlockSpec((B, D), lambda i: (0, 0)),
        ],
        out_shape=[
            jax.ShapeDtypeStruct((N_NODES, L), jnp.float32),
            jax.ShapeDtypeStruct((B, D), jnp.float32),
        ],
        scratch_shapes=[
            pltpu.VMEM((PHA, PART_CHANNELS, BB * NUM_PARTS, L), jnp.float32),
            pltpu.VMEM((B, D), jnp.float32),
            pltpu.VMEM((GCN_DIM, N_NODES), jnp.float32),
        ],
    )(x, decision_masks, W1bd, W2bd, ws_cols, bs_row, Wpe, e32, bp_row,
      params['fc_W'].T, r2(params['fc_b']), ei, eit, ews, ewst,
      b0['W1'], r2(b0['b1']), r2(b0['g1']), r2(b0['be1']),
      b0['W2'], r2(b0['b2']), r2(b0['g2']), r2(b0['be2']),
      b1['W1'], r2(b1['b1']), r2(b1['g1']), r2(b1['be1']),
      b1['W2'], r2(b1['b2']), r2(b1['g2']), r2(b1['be2']),
      b1['Wd'], r2(b1['gd']), r2(b1['bd']))

    parts_masks = masks_t.reshape(B, NUM_PARTS, SIZE, SIZE)
    return out, parts_masks


# speedup vs baseline: 1.3348x; 1.3348x over previous
import math

import jax
import jax.numpy as jnp
from jax.experimental import pallas as pl
from jax.experimental.pallas import tpu as pltpu

B = 16
L = 576
SIZE = 24
D = 768
NUM_PARTS = 4
PARTS_DIM = D // NUM_PARTS
PART_CHANNELS = 32
GCN_DIM = 392
NUM_EDGES = NUM_PARTS * (NUM_PARTS - 1) * B
N_NODES = B * NUM_PARTS
BB = 2
PHA = B // BB
HID = PARTS_DIM // 16
CPS = 4
NKB = PART_CHANNELS // CPS
NTOT = PHA + NKB
PC4 = NUM_PARTS * PART_CHANNELS

_INV_SQRT2 = 1.0 / math.sqrt(2.0)


def _gelu(t):
    return 0.5 * t * (1.0 + jax.lax.erf(t * _INV_SQRT2))


def _ln(t, g, b, eps=1e-6):
    m = jnp.mean(t, axis=-1, keepdims=True)
    v = jnp.mean((t - m) ** 2, axis=-1, keepdims=True)
    return (t - m) / jnp.sqrt(v + eps) * g + b


def _dot(a, b):
    return jnp.dot(a, b, preferred_element_type=jnp.float32)


def _build_A(ei_ref, ew_ref):
    ew_row = jax.nn.sigmoid(ew_ref[...])[None, :]
    dst_row = ei_ref[1:2, :]
    src_row = ei_ref[0:1, :]
    dst_col = dst_row.T
    row_ids = jax.lax.broadcasted_iota(jnp.int32, (N_NODES, NUM_EDGES), 0)
    col_ids = jax.lax.broadcasted_iota(jnp.int32, (NUM_EDGES, N_NODES), 1)
    Mdst = jnp.where(row_ids == dst_row, 1.0, 0.0)
    MdstT = jnp.where(col_ids == dst_col, 1.0, 0.0)
    Msrc = jnp.where(row_ids == src_row, 1.0, 0.0).T
    deg_row = _dot(ew_row, MdstT) + 1.0
    dis_row = jnp.where(deg_row > 0,
                        jax.lax.rsqrt(jnp.maximum(deg_row, 1e-12)), 0.0)
    dis_col = dis_row.T
    W_raw = _dot(Mdst * ew_row, Msrc)
    ir = jax.lax.broadcasted_iota(jnp.int32, (N_NODES, N_NODES), 0)
    ic = jax.lax.broadcasted_iota(jnp.int32, (N_NODES, N_NODES), 1)
    eye = jnp.where(ir == ic, 1.0, 0.0)
    return dis_col * (W_raw + eye) * dis_row


def _fused_body(x_ref, dm_ref, W1bd_ref, W2bd_ref, ws_ref, bs_ref, Wpe_ref,
                e32_ref, bp_ref, fw_ref, fb_ref, ei_ref, ew0_ref, ew1_ref,
                W1a_ref, b1a_ref, g1a_ref, be1a_ref,
                W2a_ref, b2a_ref, g2a_ref, be2a_ref,
                W1b_ref, b1b_ref, g1b_ref, be1b_ref,
                W2b_ref, b2b_ref, g2b_ref, be2b_ref,
                Wd_ref, gd_ref, bd_ref,
                m_ref, out_ref, x3_scr, sh_scr, acc_ref):
    i = pl.program_id(0)

    @pl.when(i == 0)
    def _():
        acc_ref[...] = jnp.zeros_like(acc_ref)

    @pl.when(i < PHA)
    def _():
        for bb in range(BB):
            xf = x_ref[bb]
            dmv = dm_ref[bb].T
            avg_all = jnp.mean(xf, axis=0, keepdims=True)
            mx_all = jnp.max(xf, axis=0, keepdims=True)
            ha = jnp.maximum(_dot(avg_all, W1bd_ref[...]), 0.0)
            hm = jnp.maximum(_dot(mx_all, W1bd_ref[...]), 0.0)
            gate = jax.nn.sigmoid(_dot(ha, W2bd_ref[...])
                                  + _dot(hm, W2bd_ref[...]))
            xg = xf * gate
            z = _dot(xg, Wpe_ref[...])
            savg4 = z[:, PC4:]
            smax4 = jnp.concatenate(
                [jnp.max(xg[:, p * PARTS_DIM:(p + 1) * PARTS_DIM],
                         axis=1, keepdims=True) for p in range(NUM_PARTS)],
                axis=1)
            sm4 = jax.nn.sigmoid(savg4 * ws_ref[0:1] + smax4 * ws_ref[1:2]
                                 + dmv * ws_ref[2:3] + bs_ref[...])
            sm128 = _dot(sm4, e32_ref[...])
            xo = z[:, :PC4] * sm128 + bp_ref[...]
            xoT = xo.T
            for p in range(NUM_PARTS):
                x3_scr[i, :, bb * NUM_PARTS + p, :] = (
                    xoT[p * PART_CHANNELS:(p + 1) * PART_CHANNELS])
            m_ref[bb] = sm4.T.reshape(NUM_PARTS, SIZE, SIZE)
            sh_scr[pl.ds(i * BB + bb, 1)] = avg_all

    @pl.when(i >= PHA)
    def _():
        c0 = (i - PHA) * CPS
        xgT = jnp.concatenate(
            [jnp.concatenate([x3_scr[s, c0 + j] for s in range(PHA)],
                             axis=0).T for j in range(CPS)], axis=0)
        acc_ref[...] += _dot(fw_ref[...], xgT)

    @pl.when(i == NTOT - 1)
    def _():
        y = acc_ref[...].T + fb_ref[...][None, :]
        A1 = _build_A(ei_ref, ew0_ref)
        A2 = _build_A(ei_ref, ew1_ref)

        h = _dot(A1, _dot(y, W1a_ref[...])) + b1a_ref[...]
        h = _gelu(_ln(h, g1a_ref[...], be1a_ref[...]))
        h = _dot(A1, _dot(h, W2a_ref[...])) + b2a_ref[...]
        y1 = _gelu(_ln(h, g2a_ref[...], be2a_ref[...]) + y)

        h = _dot(A2, _dot(y1, W1b_ref[...])) + b1b_ref[...]
        h = _gelu(_ln(h, g1b_ref[...], be1b_ref[...]))
        h = _dot(A2, _dot(h, W2b_ref[...])) + b2b_ref[...]
        h = _ln(h, g2b_ref[...], be2b_ref[...])
        sc = _ln(_dot(y1, Wd_ref[...]), gd_ref[...], bd_ref[...])
        y2 = _gelu(h + sc)

        pr = jax.lax.broadcasted_iota(jnp.int32, (B, N_NODES), 0)
        pc = jax.lax.broadcasted_iota(jnp.int32, (B, N_NODES), 1)
        pool = jnp.where(pc // NUM_PARTS == pr, 0.25, 0.0)
        out_ref[...] = _dot(pool, y2) + sh_scr[...]


@jax.jit
def kernel(decision_masks, x, params, edge_index):
    cb = params['cbam']
    W1bd = jax.scipy.linalg.block_diag(*[c['W1'] for c in cb])
    W2bd = jax.scipy.linalg.block_diag(*[c['W2'] for c in cb])
    Wpbd = jax.scipy.linalg.block_diag(*[c['Wp'] for c in cb])
    part_of_d = jnp.arange(D, dtype=jnp.int32) // PARTS_DIM
    bandmask = jnp.where(part_of_d[:, None]
                         == jnp.arange(NUM_PARTS, dtype=jnp.int32)[None, :],
                         1.0 / PARTS_DIM, 0.0)
    Wpe = jnp.concatenate([Wpbd, bandmask], axis=1)
    part_of_c = jnp.arange(PC4, dtype=jnp.int32) // PART_CHANNELS
    e32 = jnp.where(jnp.arange(NUM_PARTS, dtype=jnp.int32)[:, None]
                    == part_of_c[None, :], 1.0, 0.0)
    bp_row = jnp.concatenate([c['bp'] for c in cb]).reshape(1, PC4)
    ws_cols = jnp.stack([c['Ws'] for c in cb], axis=1)
    bs_row = jnp.stack([c['bs'] for c in cb]).reshape(1, NUM_PARTS)

    blocks = params['blocks']
    ei = edge_index.astype(jnp.int32)
    b0, b1 = blocks

    full = lambda s: pl.BlockSpec(s, lambda i: tuple(0 for _ in s))

    parts_masks, out = pl.pallas_call(
        _fused_body,
        grid=(NTOT,),
        in_specs=[
            pl.BlockSpec((BB, L, D), lambda i: (jnp.minimum(i, PHA - 1), 0, 0)),
            pl.BlockSpec((BB, 1, L), lambda i: (jnp.minimum(i, PHA - 1), 0, 0)),
            full((D, NUM_PARTS * HID)),
            full((NUM_PARTS * HID, D)),
            full((3, NUM_PARTS)),
            full((1, NUM_PARTS)),
            full((D, PC4 + NUM_PARTS)),
            full((NUM_PARTS, PC4)),
            full((1, PC4)),
            pl.BlockSpec((GCN_DIM, CPS * L),
                         lambda i: (0, jnp.maximum(i - PHA, 0))),
            full((GCN_DIM,)),
            full((2, NUM_EDGES)),
            full((NUM_EDGES,)),
            full((NUM_EDGES,)),
            full(b0['W1'].shape), full(b0['b1'].shape),
            full(b0['g1'].shape), full(b0['be1'].shape),
            full(b0['W2'].shape), full(b0['b2'].shape),
            full(b0['g2'].shape), full(b0['be2'].shape),
            full(b1['W1'].shape), full(b1['b1'].shape),
            full(b1['g1'].shape), full(b1['be1'].shape),
            full(b1['W2'].shape), full(b1['b2'].shape),
            full(b1['g2'].shape), full(b1['be2'].shape),
            full(b1['Wd'].shape), full(b1['gd'].shape),
            full(b1['bd'].shape),
        ],
        out_specs=[
            pl.BlockSpec((BB, NUM_PARTS, SIZE, SIZE),
                         lambda i: (jnp.minimum(i, PHA - 1), 0, 0, 0)),
            pl.BlockSpec((B, D), lambda i: (0, 0)),
        ],
        out_shape=[
            jax.ShapeDtypeStruct((B, NUM_PARTS, SIZE, SIZE), jnp.float32),
            jax.ShapeDtypeStruct((B, D), jnp.float32),
        ],
        scratch_shapes=[
            pltpu.VMEM((PHA, PART_CHANNELS, BB * NUM_PARTS, L), jnp.float32),
            pltpu.VMEM((B, D), jnp.float32),
            pltpu.VMEM((GCN_DIM, N_NODES), jnp.float32),
        ],
    )(x, decision_masks.transpose(0, 2, 1), W1bd, W2bd, ws_cols, bs_row,
      Wpe, e32, bp_row,
      params['fc_W'].T, params['fc_b'], ei,
      b0['edge_weight'], b1['edge_weight'],
      b0['W1'], b0['b1'], b0['g1'], b0['be1'],
      b0['W2'], b0['b2'], b0['g2'], b0['be2'],
      b1['W1'], b1['b1'], b1['g1'], b1['be1'],
      b1['W2'], b1['b2'], b1['g2'], b1['be2'],
      b1['Wd'], b1['gd'], b1['bd'])

    return out, parts_masks


# speedup vs baseline: 1.4194x; 1.0633x over previous
import math

import jax
import jax.numpy as jnp
from jax.experimental import pallas as pl
from jax.experimental.pallas import tpu as pltpu

B = 16
L = 576
SIZE = 24
D = 768
NUM_PARTS = 4
PARTS_DIM = D // NUM_PARTS
PART_CHANNELS = 32
GCN_DIM = 392
NUM_EDGES = NUM_PARTS * (NUM_PARTS - 1) * B
N_NODES = B * NUM_PARTS
BB = 2
PHA = B // BB
HID = PARTS_DIM // 16
CPS = 4
NKB = PART_CHANNELS // CPS
NTOT = PHA + NKB
PC4 = NUM_PARTS * PART_CHANNELS

_INV_SQRT2 = 1.0 / math.sqrt(2.0)


def _gelu(t):
    return 0.5 * t * (1.0 + jax.lax.erf(t * _INV_SQRT2))


def _ln(t, g, b, eps=1e-6):
    m = jnp.mean(t, axis=-1, keepdims=True)
    v = jnp.mean((t - m) ** 2, axis=-1, keepdims=True)
    return (t - m) / jnp.sqrt(v + eps) * g + b


def _dot(a, b):
    return jnp.dot(a, b, preferred_element_type=jnp.float32)


def _build_A(ei_ref, ew_ref):
    ew_row = jax.nn.sigmoid(ew_ref[...])[None, :]
    dst_row = ei_ref[1:2, :]
    src_row = ei_ref[0:1, :]
    dst_col = dst_row.T
    row_ids = jax.lax.broadcasted_iota(jnp.int32, (N_NODES, NUM_EDGES), 0)
    col_ids = jax.lax.broadcasted_iota(jnp.int32, (NUM_EDGES, N_NODES), 1)
    Mdst = jnp.where(row_ids == dst_row, 1.0, 0.0)
    MdstT = jnp.where(col_ids == dst_col, 1.0, 0.0)
    Msrc = jnp.where(row_ids == src_row, 1.0, 0.0).T
    deg_row = _dot(ew_row, MdstT) + 1.0
    dis_row = jnp.where(deg_row > 0,
                        jax.lax.rsqrt(jnp.maximum(deg_row, 1e-12)), 0.0)
    dis_col = dis_row.T
    W_raw = _dot(Mdst * ew_row, Msrc)
    ir = jax.lax.broadcasted_iota(jnp.int32, (N_NODES, N_NODES), 0)
    ic = jax.lax.broadcasted_iota(jnp.int32, (N_NODES, N_NODES), 1)
    eye = jnp.where(ir == ic, 1.0, 0.0)
    return dis_col * (W_raw + eye) * dis_row


def _fused_body(x_ref, dm_ref, W1T_refs, W2T_refs, WpT_refs, bp_refs,
                ws_ref, bs_ref, fw_ref, fb_ref, ei_ref, ew0_ref, ew1_ref,
                W1a_ref, b1a_ref, g1a_ref, be1a_ref,
                W2a_ref, b2a_ref, g2a_ref, be2a_ref,
                W1b_ref, b1b_ref, g1b_ref, be1b_ref,
                W2b_ref, b2b_ref, g2b_ref, be2b_ref,
                Wd_ref, gd_ref, bd_ref,
                m_ref, out_ref, x3_scr, sh_scr, acc_ref,
                W1bd_scr, W2bd_scr, Wpe_scr, bp_scr):
    i = pl.program_id(0)

    @pl.when(i == 0)
    def _():
        acc_ref[...] = jnp.zeros_like(acc_ref)
        W1bd_scr[...] = jnp.zeros_like(W1bd_scr)
        W2bd_scr[...] = jnp.zeros_like(W2bd_scr)
        Wpe_scr[...] = jnp.zeros_like(Wpe_scr)
        pd = jax.lax.broadcasted_iota(jnp.int32, (D, NUM_PARTS), 0)
        pc = jax.lax.broadcasted_iota(jnp.int32, (D, NUM_PARTS), 1)
        Wpe_scr[:, PC4:] = jnp.where(pd // PARTS_DIM == pc,
                                     1.0 / PARTS_DIM, 0.0)
        for p in range(NUM_PARTS):
            r0, r1 = p * PARTS_DIM, (p + 1) * PARTS_DIM
            W1bd_scr[r0:r1, p * HID:(p + 1) * HID] = W1T_refs[p][...].T
            W2bd_scr[p * HID:(p + 1) * HID, r0:r1] = W2T_refs[p][...].T
            Wpe_scr[r0:r1, p * PART_CHANNELS:(p + 1) * PART_CHANNELS] = (
                WpT_refs[p][...].T)
            bp_scr[0:1, p * PART_CHANNELS:(p + 1) * PART_CHANNELS] = (
                bp_refs[p][...][None, :])

    @pl.when(i < PHA)
    def _():
        for bb in range(BB):
            xf = x_ref[bb]
            dmv = dm_ref[bb].T
            avg_all = jnp.mean(xf, axis=0, keepdims=True)
            mx_all = jnp.max(xf, axis=0, keepdims=True)
            ha = jnp.maximum(_dot(avg_all, W1bd_scr[...]), 0.0)
            hm = jnp.maximum(_dot(mx_all, W1bd_scr[...]), 0.0)
            gate = jax.nn.sigmoid(_dot(ha, W2bd_scr[...])
                                  + _dot(hm, W2bd_scr[...]))
            xg = xf * gate
            z = _dot(xg, Wpe_scr[...])
            savg4 = z[:, PC4:]
            smax4 = jnp.concatenate(
                [jnp.max(xg[:, p * PARTS_DIM:(p + 1) * PARTS_DIM],
                         axis=1, keepdims=True) for p in range(NUM_PARTS)],
                axis=1)
            sm4 = jax.nn.sigmoid(savg4 * ws_ref[0:1] + smax4 * ws_ref[1:2]
                                 + dmv * ws_ref[2:3]
                                 + bs_ref[...][None, :])
            e32 = jnp.where(
                jax.lax.broadcasted_iota(jnp.int32, (NUM_PARTS, PC4), 0)
                == jax.lax.broadcasted_iota(jnp.int32, (NUM_PARTS, PC4), 1)
                // PART_CHANNELS, 1.0, 0.0)
            sm128 = _dot(sm4, e32)
            xo = z[:, :PC4] * sm128 + bp_scr[...]
            xoT = xo.T
            for p in range(NUM_PARTS):
                x3_scr[i, :, bb * NUM_PARTS + p, :] = (
                    xoT[p * PART_CHANNELS:(p + 1) * PART_CHANNELS])
            m_ref[bb] = sm4.T.reshape(NUM_PARTS, SIZE, SIZE)
            sh_scr[pl.ds(i * BB + bb, 1)] = avg_all

    @pl.when(i >= PHA)
    def _():
        c0 = (i - PHA) * CPS
        xgT = jnp.concatenate(
            [jnp.concatenate([x3_scr[s, c0 + j] for s in range(PHA)],
                             axis=0).T for j in range(CPS)], axis=0)
        acc_ref[...] += _dot(fw_ref[...], xgT)

    @pl.when(i == NTOT - 1)
    def _():
        y = acc_ref[...].T + fb_ref[...][None, :]
        A1 = _build_A(ei_ref, ew0_ref)
        A2 = _build_A(ei_ref, ew1_ref)

        h = _dot(A1, _dot(y, W1a_ref[...])) + b1a_ref[...]
        h = _gelu(_ln(h, g1a_ref[...], be1a_ref[...]))
        h = _dot(A1, _dot(h, W2a_ref[...])) + b2a_ref[...]
        y1 = _gelu(_ln(h, g2a_ref[...], be2a_ref[...]) + y)

        h = _dot(A2, _dot(y1, W1b_ref[...])) + b1b_ref[...]
        h = _gelu(_ln(h, g1b_ref[...], be1b_ref[...]))
        h = _dot(A2, _dot(h, W2b_ref[...])) + b2b_ref[...]
        h = _ln(h, g2b_ref[...], be2b_ref[...])
        sc = _ln(_dot(y1, Wd_ref[...]), gd_ref[...], bd_ref[...])
        y2 = _gelu(h + sc)

        pr = jax.lax.broadcasted_iota(jnp.int32, (B, N_NODES), 0)
        pc = jax.lax.broadcasted_iota(jnp.int32, (B, N_NODES), 1)
        pool = jnp.where(pc // NUM_PARTS == pr, 0.25, 0.0)
        out_ref[...] = _dot(pool, y2) + sh_scr[...]


@jax.jit
def kernel(decision_masks, x, params, edge_index):
    cb = params['cbam']
    ws_cols = jnp.stack([c['Ws'] for c in cb], axis=1)
    bs4 = jnp.stack([c['bs'] for c in cb])

    blocks = params['blocks']
    ei = edge_index.astype(jnp.int32)
    b0, b1 = blocks

    full = lambda s: pl.BlockSpec(s, lambda i: tuple(0 for _ in s))

    parts_masks, out = pl.pallas_call(
        _fused_body,
        grid=(NTOT,),
        in_specs=[
            pl.BlockSpec((BB, L, D), lambda i: (jnp.minimum(i, PHA - 1), 0, 0)),
            pl.BlockSpec((BB, 1, L), lambda i: (jnp.minimum(i, PHA - 1), 0, 0)),
            tuple(full((HID, PARTS_DIM)) for _ in range(NUM_PARTS)),
            tuple(full((PARTS_DIM, HID)) for _ in range(NUM_PARTS)),
            tuple(full((PART_CHANNELS, PARTS_DIM)) for _ in range(NUM_PARTS)),
            tuple(full((PART_CHANNELS,)) for _ in range(NUM_PARTS)),
            full((3, NUM_PARTS)),
            full((NUM_PARTS,)),
            pl.BlockSpec((GCN_DIM, CPS * L),
                         lambda i: (0, jnp.maximum(i - PHA, 0))),
            full((GCN_DIM,)),
            full((2, NUM_EDGES)),
            full((NUM_EDGES,)),
            full((NUM_EDGES,)),
            full(b0['W1'].shape), full(b0['b1'].shape),
            full(b0['g1'].shape), full(b0['be1'].shape),
            full(b0['W2'].shape), full(b0['b2'].shape),
            full(b0['g2'].shape), full(b0['be2'].shape),
            full(b1['W1'].shape), full(b1['b1'].shape),
            full(b1['g1'].shape), full(b1['be1'].shape),
            full(b1['W2'].shape), full(b1['b2'].shape),
            full(b1['g2'].shape), full(b1['be2'].shape),
            full(b1['Wd'].shape), full(b1['gd'].shape),
            full(b1['bd'].shape),
        ],
        out_specs=[
            pl.BlockSpec((BB, NUM_PARTS, SIZE, SIZE),
                         lambda i: (jnp.minimum(i, PHA - 1), 0, 0, 0)),
            pl.BlockSpec((B, D), lambda i: (0, 0)),
        ],
        out_shape=[
            jax.ShapeDtypeStruct((B, NUM_PARTS, SIZE, SIZE), jnp.float32),
            jax.ShapeDtypeStruct((B, D), jnp.float32),
        ],
        scratch_shapes=[
            pltpu.VMEM((PHA, PART_CHANNELS, BB * NUM_PARTS, L), jnp.float32),
            pltpu.VMEM((B, D), jnp.float32),
            pltpu.VMEM((GCN_DIM, N_NODES), jnp.float32),
            pltpu.VMEM((D, NUM_PARTS * HID), jnp.float32),
            pltpu.VMEM((NUM_PARTS * HID, D), jnp.float32),
            pltpu.VMEM((D, PC4 + NUM_PARTS), jnp.float32),
            pltpu.VMEM((1, PC4), jnp.float32),
        ],
    )(x, decision_masks.transpose(0, 2, 1),
      tuple(c['W1'].T for c in cb),
      tuple(c['W2'].T for c in cb),
      tuple(c['Wp'].T for c in cb),
      tuple(c['bp'] for c in cb),
      ws_cols, bs4,
      params['fc_W'].T, params['fc_b'], ei,
      b0['edge_weight'], b1['edge_weight'],
      b0['W1'], b0['b1'], b0['g1'], b0['be1'],
      b0['W2'], b0['b2'], b0['g2'], b0['be2'],
      b1['W1'], b1['b1'], b1['g1'], b1['be1'],
      b1['W2'], b1['b2'], b1['g2'], b1['be2'],
      b1['Wd'], b1['gd'], b1['bd'])

    return out, parts_masks
